# baseline (speedup 1.0000x reference)
import functools

import jax
import jax.numpy as jnp
from jax import lax
from jax.experimental import pallas as pl
from jax.experimental.pallas import tpu as pltpu
from jax.experimental.pallas import tpu_sc as plsc

BATCH = 16384
DIM = 64
NUM_CORES = 2
NUM_SUBCORES = 16
NW = NUM_CORES * NUM_SUBCORES
BPW = BATCH // NW
CHUNK = 128
NCHUNK = BPW // CHUNK
LANES = 16
CPR = DIM // LANES


def _gmf_body(uid_hbm, iid_hbm, utab_hbm, itab_hbm, out_hbm,
              uidx_v, iidx_v, urows_v, irows_v, sem):
    c = lax.axis_index("c")
    s = lax.axis_index("s")
    wid = s * NUM_CORES + c
    base = wid * BPW

    pltpu.sync_copy(uid_hbm.at[pl.ds(base, BPW)], uidx_v)
    pltpu.sync_copy(iid_hbm.at[pl.ds(base, BPW)], iidx_v)

    copies = []
    for j in range(NCHUNK):
        rows = pl.ds(j * CHUNK, CHUNK)
        copies.append(pltpu.async_copy(
            utab_hbm.at[uidx_v.at[rows]], urows_v.at[rows], sem))
        copies.append(pltpu.async_copy(
            itab_hbm.at[iidx_v.at[rows]], irows_v.at[rows], sem))
    for cp in copies:
        cp.wait()

    def body(r, carry):
        for ci in range(CPR):
            sl = pl.ds(ci * LANES, LANES)
            urows_v[r, sl] = urows_v[r, sl] * irows_v[r, sl]
        return carry

    lax.fori_loop(0, BPW, body, 0)

    pltpu.sync_copy(urows_v, out_hbm.at[pl.ds(base, BPW)])


@jax.jit
def _gmf(user_id, item_id, mf_user_emb, mf_item_emb):
    mesh = plsc.VectorSubcoreMesh(core_axis_name="c", subcore_axis_name="s")
    f = pl.kernel(
        _gmf_body,
        mesh=mesh,
        compiler_params=pltpu.CompilerParams(use_tc_tiling_on_sc=False),
        out_type=jax.ShapeDtypeStruct((BATCH, DIM), jnp.float32),
        scratch_types=[
            pltpu.VMEM((BPW,), jnp.int32),
            pltpu.VMEM((BPW,), jnp.int32),
            pltpu.VMEM((BPW, DIM), jnp.float32),
            pltpu.VMEM((BPW, DIM), jnp.float32),
            pltpu.SemaphoreType.DMA,
        ],
    )
    return f(user_id, item_id, mf_user_emb, mf_item_emb)


def kernel(user_id, item_id, mf_user_emb, mf_item_emb):
    return _gmf(user_id.astype(jnp.int32), item_id.astype(jnp.int32),
                mf_user_emb, mf_item_emb)


# speedup vs baseline: 1.9955x; 1.9955x over previous
import functools

import jax
import jax.numpy as jnp
from jax import lax
from jax.experimental import pallas as pl
from jax.experimental.pallas import tpu as pltpu
from jax.experimental.pallas import tpu_sc as plsc

BATCH = 16384
DIM = 64
GRP = 8
NUM_CORES = 2
NUM_SUBCORES = 16
NW = NUM_CORES * NUM_SUBCORES
BPW = BATCH // NW
LANES = 16
CHUNK = LANES
NCH = BPW // CHUNK
CPR = DIM // LANES


def _gmf_body(uid_hbm, iid_hbm, utab_hbm, itab_hbm, out_hbm,
              uidx_v, iidx_v, tilebuf_v, out2d_v, sem):
    c = lax.axis_index("c")
    s = lax.axis_index("s")
    wid = s * NUM_CORES + c
    base = wid * BPW

    pltpu.sync_copy(uid_hbm.at[pl.ds(base, BPW)], uidx_v)
    pltpu.sync_copy(iid_hbm.at[pl.ds(base, BPW)], iidx_v)

    def gather_chunks(tab_hbm, idx_v, extract):
        def chunk(k, carry):
            vec = idx_v[pl.ds(k * CHUNK, CHUNK)]
            gvec = lax.shift_right_logical(vec, 3)
            for l in range(CHUNK):
                pltpu.async_copy(tab_hbm.at[gvec[l]], tilebuf_v.at[l], sem)
            pltpu.make_async_copy(tab_hbm.at[pl.ds(0, CHUNK)],
                                  tilebuf_v, sem).wait()
            rvec = lax.bitwise_and(vec, 7)
            for l in range(CHUNK):
                extract(k * CHUNK + l, l, rvec[l])
            return carry

        lax.fori_loop(0, NCH, chunk, 0)

    def take_user(dst, j, r):
        for ci in range(CPR):
            sl = pl.ds(ci * LANES, LANES)
            out2d_v[dst, sl] = tilebuf_v[j, r, sl]

    gather_chunks(utab_hbm, uidx_v, take_user)

    def take_item(dst, j, r):
        for ci in range(CPR):
            sl = pl.ds(ci * LANES, LANES)
            out2d_v[dst, sl] = out2d_v[dst, sl] * tilebuf_v[j, r, sl]

    gather_chunks(itab_hbm, iidx_v, take_item)

    pltpu.sync_copy(out2d_v, out_hbm.at[pl.ds(base, BPW)])


@jax.jit
def _gmf(user_id, item_id, mf_user_emb, mf_item_emb):
    utab3 = mf_user_emb.reshape(-1, GRP, DIM)
    itab3 = mf_item_emb.reshape(-1, GRP, DIM)
    mesh = plsc.VectorSubcoreMesh(core_axis_name="c", subcore_axis_name="s")
    f = pl.kernel(
        _gmf_body,
        mesh=mesh,
        compiler_params=pltpu.CompilerParams(use_tc_tiling_on_sc=True),
        out_type=jax.ShapeDtypeStruct((BATCH, DIM), jnp.float32),
        scratch_types=[
            pltpu.VMEM((BPW,), jnp.int32),
            pltpu.VMEM((BPW,), jnp.int32),
            pltpu.VMEM((CHUNK, GRP, DIM), jnp.float32),
            pltpu.VMEM((BPW, DIM), jnp.float32),
            pltpu.SemaphoreType.DMA,
        ],
    )
    return f(user_id, item_id, utab3, itab3)


def kernel(user_id, item_id, mf_user_emb, mf_item_emb):
    return _gmf(user_id.astype(jnp.int32), item_id.astype(jnp.int32),
                mf_user_emb, mf_item_emb)
